# baseline (speedup 1.0000x reference)
import jax
import jax.numpy as jnp
from jax import lax
from jax.experimental import pallas as pl
from jax.experimental.pallas import tpu as pltpu
from jax.experimental.pallas import tpu_sc as plsc

H = 512
W = 512
TROWS = 9
RB = 8 * TROWS
NLEV = 2
NTCLEV = 3 - NLEV
NPAN = 4
NBATCH = 4
NSTRIP = 8
NW = 32


def _sc_partials_body(pre_hbm, part_hbm, buf, obuf, sem0, sem1):
    cid = lax.axis_index("c")
    sid = lax.axis_index("s")
    wid = sid * 2 + cid
    n = wid // NSTRIP
    strip = wid % NSTRIP
    row0 = pl.multiple_of(jnp.minimum(strip * 64, H - RB), 8)
    rr0 = jnp.where(strip == NSTRIP - 1, 9, 1)
    rr_end = jnp.where(strip == NSTRIP - 1, 71, 65)

    sems = (sem0, sem1)

    def pbase(level, m):
        return (level * NPAN + m) * RB

    def start(level):
        return [
            pltpu.async_copy(
                pre_hbm.at[level, n, 2, pl.ds(row0, RB),
                           pl.ds(128 * m, 128)],
                buf.at[pl.ds(pbase(level, m), RB), :],
                sems[level],
            )
            for m in range(NPAN)
        ]

    cps = [start(level) for level in range(NLEV)]

    iota = lax.iota(jnp.int32, 16)
    lanef = iota.astype(jnp.float32)
    zero = jnp.zeros((16,), jnp.float32)
    one = jnp.full((16,), 1.0, jnp.float32)
    base_rowf = row0.astype(jnp.float32)
    rr0f = rr0.astype(jnp.float32)
    shr_idx = jnp.maximum(iota - 1, 0)
    shl_idx = jnp.minimum(iota + 1, 15)
    bc0_idx = jnp.zeros((16,), jnp.int32)
    bc15_idx = jnp.full((16,), 15, jnp.int32)

    def perm(v, idx):
        return jnp.take_along_axis(v, idx, axis=0, mode="promise_in_bounds")

    def sweep_chunk(pb, c, colmask, gcf, accs, edge):
        a_cnt0, a_i0, a_jb0 = accs
        eside = edge[0] if edge is not None else None
        epb = edge[1] if edge is not None else None

        def ldrow(r):
            cc_ = buf[r, pl.ds(c, 16)]
            if eside == "l":
                sh = perm(cc_, shr_idx)
                if epb is None:
                    lf = sh
                else:
                    ev = perm(buf[epb + r - pb, pl.ds(112, 16)], bc15_idx)
                    lf = jnp.where(iota == 0, ev, sh)
            else:
                lf = buf[r, pl.ds(c - 1, 16)]
            if eside == "r":
                sh = perm(cc_, shl_idx)
                if epb is None:
                    rt = sh
                else:
                    ev = perm(buf[epb + r - pb, pl.ds(0, 16)], bc0_idx)
                    rt = jnp.where(iota == 15, ev, sh)
            else:
                rt = buf[r, pl.ds(c + 1, 16)]
            return lf, cc_, rt

        r_prev = pb + rr0 - 1
        r_cur = pb + rr0
        p_l, p_c, p_r = ldrow(r_prev)
        hm3_prev = jnp.maximum(jnp.maximum(p_l, p_r), p_c)
        c_l, c_c, c_r = ldrow(r_cur)
        hm2_cur = jnp.maximum(c_l, c_r)
        hm3_cur = jnp.maximum(hm2_cur, c_c)

        init = (hm3_prev, hm3_cur, hm2_cur, c_c,
                base_rowf + rr0f, a_cnt0, a_i0, a_jb0)

        @plsc.parallel_loop(rr0, rr_end, step=2, unroll=4, carry=init)
        def _rows(rr, carry):
            h3p, h3c, h2c, cc, r1f, a_cnt, a_i, a_jb = carry
            n1l, n1c, n1r = ldrow(pb + rr + 1)
            n2l, n2c, n2r = ldrow(pb + rr + 2)
            hm2_n1 = jnp.maximum(n1l, n1r)
            hm3_n1 = jnp.maximum(hm2_n1, n1c)
            hm2_n2 = jnp.maximum(n2l, n2r)
            hm3_n2 = jnp.maximum(hm2_n2, n2c)
            mx1 = jnp.maximum(jnp.maximum(h3p, hm3_n1), h2c)
            mx2 = jnp.maximum(jnp.maximum(h3c, hm3_n2), hm2_n1)
            mf1 = jnp.where(cc > mx1, colmask, zero)
            mf2 = jnp.where(n1c > mx2, colmask, zero)
            mfs = mf1 + mf2
            a_cnt = a_cnt + mfs
            a_i = a_i + (mf1 * r1f + mf2 * (r1f + 1.0))
            a_jb = a_jb + mfs * gcf
            return (hm3_n1, hm3_n2, hm2_n2, n2c,
                    r1f + 2.0, a_cnt, a_i, a_jb)

        return (_rows[5], _rows[6], _rows[7])

    lmask = jnp.where(iota >= 1, one, zero)
    rmask = jnp.where(iota <= 14, one, zero)

    SIv = zero
    SJv = zero
    CNTv = zero
    for level in range(NLEV):
        for cp in cps[level]:
            cp.wait()

        accs = (zero, zero, zero)
        for m in range(NPAN):
            pb = pbase(level, m)
            gc0 = jnp.float32(128 * m)
            accs = sweep_chunk(
                pb, 0, lmask if m == 0 else one, gc0, accs,
                ("l", None if m == 0 else pbase(level, m - 1)))

            def chunk_body(j, carry, pb=pb, m=m):
                cj = j * 16
                gcf = jnp.float32(128 * m) + cj.astype(jnp.float32)
                return sweep_chunk(pb, cj, one, gcf, carry, None)

            accs = lax.fori_loop(1, 7, chunk_body, accs)

            accs = sweep_chunk(
                pb, 112, rmask if m == NPAN - 1 else one,
                gc0 + 112.0, accs,
                ("r", None if m == NPAN - 1 else pbase(level, m + 1)))

        a_cnt, a_i, a_jb = accs
        R = jnp.float32(4.0 * (2 ** level))
        SIv = SIv + R * a_i
        SJv = SJv + R * (a_jb + lanef * a_cnt)
        CNTv = CNTv + a_cnt

    obuf[pl.ds(0, 16)] = SIv
    obuf[pl.ds(16, 16)] = SJv
    obuf[pl.ds(32, 16)] = CNTv
    for q in range(3, 8):
        obuf[pl.ds(16 * q, 16)] = zero
    pltpu.sync_copy(obuf, part_hbm.at[wid, :])


def _make_sc_partials():
    mesh = plsc.VectorSubcoreMesh(
        core_axis_name="c", subcore_axis_name="s", num_cores=2, num_subcores=16
    )
    return pl.kernel(
        _sc_partials_body,
        out_type=jax.ShapeDtypeStruct((NW, 128), jnp.float32),
        mesh=mesh,
        scratch_types=[
            pltpu.VMEM((NLEV * NPAN * RB, 128), jnp.float32),
            pltpu.VMEM((128,), jnp.float32),
            pltpu.SemaphoreType.DMA,
            pltpu.SemaphoreType.DMA,
        ],
        compiler_params=pltpu.CompilerParams(use_tc_tiling_on_sc=True),
    )


def _tc_stencil_body(h_ref, out_ref):
    lvl = pl.program_id(0) + NLEV
    R = jnp.where(lvl == 1, jnp.float32(8.0), jnp.float32(16.0))
    h = h_ref[0, 0, 0]
    c = h[1:-1, 1:-1]
    m = ((c > h[:-2, :-2]) & (c > h[:-2, 1:-1]) & (c > h[:-2, 2:])
         & (c > h[1:-1, :-2]) & (c > h[1:-1, 2:])
         & (c > h[2:, :-2]) & (c > h[2:, 1:-1]) & (c > h[2:, 2:]))
    mf = m.astype(jnp.float32)
    ii = (lax.broadcasted_iota(jnp.int32, (H - 2, W - 2), 0)
          .astype(jnp.float32) + 1.0)
    jj = (lax.broadcasted_iota(jnp.int32, (H - 2, W - 2), 1)
          .astype(jnp.float32) + 1.0)
    si = jnp.sum(mf * ii) * R
    sj = jnp.sum(mf * jj) * R
    cnt = jnp.sum(mf)
    lane = lax.broadcasted_iota(jnp.int32, (1, 1, 1, 128), 3)
    out_ref[...] = jnp.where(
        lane == 0, si, jnp.where(lane == 1, sj,
                                 jnp.where(lane == 2, cnt, 0.0)))


def _tc_stencil(pre_offset):
    return pl.pallas_call(
        _tc_stencil_body,
        grid=(NTCLEV, NBATCH),
        in_specs=[pl.BlockSpec((1, 1, 1, H, W),
                               lambda l, i: (l + NLEV, i, 2, 0, 0))],
        out_specs=pl.BlockSpec((1, 1, 1, 128), lambda l, i: (l, i, 0, 0)),
        out_shape=jax.ShapeDtypeStruct((NTCLEV, NBATCH, 1, 128),
                                       jnp.float32),
    )(pre_offset)


def _tc_combine_body(part_ref, tcp_ref, t_ref, out_ref):
    p = part_ref[...]
    tcp = tcp_ref[...]
    t = t_ref[...]
    seg = lax.broadcasted_iota(jnp.int32, (NW, 128), 1) // 16
    grp = lax.broadcasted_iota(jnp.int32, (NW, 128), 0) // NSTRIP
    tlane = lax.broadcasted_iota(jnp.int32, (NTCLEV * NBATCH, 128), 1)
    trow = lax.broadcasted_iota(jnp.int32, (NTCLEV * NBATCH, 128), 0) % NBATCH
    nrow = lax.broadcasted_iota(jnp.int32, (NBATCH, 200), 0)
    cx = (t[:, :, 0] + t[:, :, 2]) * 0.5
    cy = (t[:, :, 1] + t[:, :, 3]) * 0.5

    zero = jnp.float32(0.0)
    off_x = zero
    off_y = zero
    cs_tx = zero
    cs_ty = zero
    ts_tx = zero
    ts_ty = zero
    point_sum = zero
    for nn in range(NBATCH):
        mrow = grp == nn
        mtrow = trow == nn
        si_n = (jnp.sum(jnp.where(mrow & (seg == 0), p, 0.0))
                + jnp.sum(jnp.where(mtrow & (tlane == 0), tcp, 0.0)))
        sj_n = (jnp.sum(jnp.where(mrow & (seg == 1), p, 0.0))
                + jnp.sum(jnp.where(mtrow & (tlane == 1), tcp, 0.0)))
        c_n = (jnp.sum(jnp.where(mrow & (seg == 2), p, 0.0))
               + jnp.sum(jnp.where(mtrow & (tlane == 2), tcp, 0.0)))
        tx_n = jnp.sum(jnp.where(nrow == nn, cx, 0.0))
        ty_n = jnp.sum(jnp.where(nrow == nn, cy, 0.0))
        dx = jnp.abs(si_n - tx_n)
        dy = jnp.abs(sj_n - ty_n)
        off_x = off_x + jnp.where(dx < 1.0, 0.5 * dx * dx, dx - 0.5)
        off_y = off_y + jnp.where(dy < 1.0, 0.5 * dy * dy, dy - 0.5)
        cs_tx = cs_tx + si_n
        cs_ty = cs_ty + sj_n
        ts_tx = ts_tx + tx_n
        ts_ty = ts_ty + ty_n
        point_sum = point_sum + c_n
    loss = (off_x / jnp.abs(off_x) * (cs_tx - ts_tx)
            + off_y / jnp.abs(off_y) * (cs_ty - ts_ty)) / point_sum
    out_ref[0, 0] = loss


def _tc_combine(part, tcp, target):
    return pl.pallas_call(
        _tc_combine_body,
        out_shape=jax.ShapeDtypeStruct((1, 1), jnp.float32),
        out_specs=pl.BlockSpec(memory_space=pltpu.SMEM),
    )(part, tcp, target)


def kernel(target, pre_offset):
    part = _make_sc_partials()(pre_offset)
    tcp = _tc_stencil(pre_offset)
    loss = _tc_combine(part, tcp.reshape(NTCLEV * NBATCH, 128), target)
    return loss[0, 0]


# speedup vs baseline: 1.0035x; 1.0035x over previous
import jax
import jax.numpy as jnp
from jax import lax
from jax.experimental import pallas as pl
from jax.experimental.pallas import tpu as pltpu
from jax.experimental.pallas import tpu_sc as plsc

H = 512
W = 512
TROWS = 9
RB = 8 * TROWS
NLEV = 2
NTCLEV = 3 - NLEV
NPAN = 4
NBATCH = 4
NSTRIP = 8
NW = 32


def _sc_partials_body(pre_hbm, part_hbm, buf, obuf, sem0, sem1):
    cid = lax.axis_index("c")
    sid = lax.axis_index("s")
    wid = sid * 2 + cid
    n = wid // NSTRIP
    strip = wid % NSTRIP
    row0 = pl.multiple_of(jnp.minimum(strip * 64, H - RB), 8)
    rr0 = jnp.where(strip == NSTRIP - 1, 9, 1)
    rr_end = jnp.where(strip == NSTRIP - 1, 71, 65)

    sems = (sem0, sem1)

    def pbase(level, m):
        return (level * NPAN + m) * RB

    def start(level):
        return [
            pltpu.async_copy(
                pre_hbm.at[level, n, 2, pl.ds(row0, RB),
                           pl.ds(128 * m, 128)],
                buf.at[pl.ds(pbase(level, m), RB), :],
                sems[level],
            )
            for m in range(NPAN)
        ]

    cps = [start(level) for level in range(NLEV)]

    iota = lax.iota(jnp.int32, 16)
    lanef = iota.astype(jnp.float32)
    zero = jnp.zeros((16,), jnp.float32)
    one = jnp.full((16,), 1.0, jnp.float32)
    base_rowf = row0.astype(jnp.float32)
    rr0f = rr0.astype(jnp.float32)
    shr_idx = jnp.maximum(iota - 1, 0)
    shl_idx = jnp.minimum(iota + 1, 15)
    bc0_idx = jnp.zeros((16,), jnp.int32)
    bc15_idx = jnp.full((16,), 15, jnp.int32)

    def perm(v, idx):
        return jnp.take_along_axis(v, idx, axis=0, mode="promise_in_bounds")

    def sweep_chunk(pb, c, colmask, gcf, accs, edge):
        a_cnt0, a_i0, a_jb0 = accs
        eside = edge[0] if edge is not None else None
        epb = edge[1] if edge is not None else None

        def ldrow(r):
            cc_ = buf[r, pl.ds(c, 16)]
            if eside == "l":
                sh = perm(cc_, shr_idx)
                if epb is None:
                    lf = sh
                else:
                    ev = perm(buf[epb + r - pb, pl.ds(112, 16)], bc15_idx)
                    lf = jnp.where(iota == 0, ev, sh)
            else:
                lf = buf[r, pl.ds(c - 1, 16)]
            if eside == "r":
                sh = perm(cc_, shl_idx)
                if epb is None:
                    rt = sh
                else:
                    ev = perm(buf[epb + r - pb, pl.ds(0, 16)], bc0_idx)
                    rt = jnp.where(iota == 15, ev, sh)
            else:
                rt = buf[r, pl.ds(c + 1, 16)]
            return lf, cc_, rt

        r_prev = pb + rr0 - 1
        r_cur = pb + rr0
        p_l, p_c, p_r = ldrow(r_prev)
        hm3_prev = jnp.maximum(jnp.maximum(p_l, p_r), p_c)
        c_l, c_c, c_r = ldrow(r_cur)
        hm2_cur = jnp.maximum(c_l, c_r)
        hm3_cur = jnp.maximum(hm2_cur, c_c)

        init = (hm3_prev, hm3_cur, hm2_cur, c_c,
                base_rowf + rr0f, a_cnt0, a_i0, a_jb0)

        @plsc.parallel_loop(rr0, rr_end, step=2, unroll=2, carry=init)
        def _rows(rr, carry):
            h3p, h3c, h2c, cc, r1f, a_cnt, a_i, a_jb = carry
            n1l, n1c, n1r = ldrow(pb + rr + 1)
            n2l, n2c, n2r = ldrow(pb + rr + 2)
            hm2_n1 = jnp.maximum(n1l, n1r)
            hm3_n1 = jnp.maximum(hm2_n1, n1c)
            hm2_n2 = jnp.maximum(n2l, n2r)
            hm3_n2 = jnp.maximum(hm2_n2, n2c)
            mx1 = jnp.maximum(jnp.maximum(h3p, hm3_n1), h2c)
            mx2 = jnp.maximum(jnp.maximum(h3c, hm3_n2), hm2_n1)
            mf1 = jnp.where(cc > mx1, colmask, zero)
            mf2 = jnp.where(n1c > mx2, colmask, zero)
            mfs = mf1 + mf2
            a_cnt = a_cnt + mfs
            a_i = a_i + (mf1 * r1f + mf2 * (r1f + 1.0))
            a_jb = a_jb + mfs * gcf
            return (hm3_n1, hm3_n2, hm2_n2, n2c,
                    r1f + 2.0, a_cnt, a_i, a_jb)

        return (_rows[5], _rows[6], _rows[7])

    lmask = jnp.where(iota >= 1, one, zero)
    rmask = jnp.where(iota <= 14, one, zero)

    SIv = zero
    SJv = zero
    CNTv = zero
    for level in range(NLEV):
        for cp in cps[level]:
            cp.wait()

        accs = (zero, zero, zero)
        for m in range(NPAN):
            pb = pbase(level, m)
            gc0 = jnp.float32(128 * m)
            accs = sweep_chunk(
                pb, 0, lmask if m == 0 else one, gc0, accs,
                ("l", None if m == 0 else pbase(level, m - 1)))

            def chunk_body(j, carry, pb=pb, m=m):
                cj = j * 16
                gcf = jnp.float32(128 * m) + cj.astype(jnp.float32)
                return sweep_chunk(pb, cj, one, gcf, carry, None)

            accs = lax.fori_loop(1, 7, chunk_body, accs)

            accs = sweep_chunk(
                pb, 112, rmask if m == NPAN - 1 else one,
                gc0 + 112.0, accs,
                ("r", None if m == NPAN - 1 else pbase(level, m + 1)))

        a_cnt, a_i, a_jb = accs
        R = jnp.float32(4.0 * (2 ** level))
        SIv = SIv + R * a_i
        SJv = SJv + R * (a_jb + lanef * a_cnt)
        CNTv = CNTv + a_cnt

    obuf[pl.ds(0, 16)] = SIv
    obuf[pl.ds(16, 16)] = SJv
    obuf[pl.ds(32, 16)] = CNTv
    for q in range(3, 8):
        obuf[pl.ds(16 * q, 16)] = zero
    pltpu.sync_copy(obuf, part_hbm.at[wid, :])


def _make_sc_partials():
    mesh = plsc.VectorSubcoreMesh(
        core_axis_name="c", subcore_axis_name="s", num_cores=2, num_subcores=16
    )
    return pl.kernel(
        _sc_partials_body,
        out_type=jax.ShapeDtypeStruct((NW, 128), jnp.float32),
        mesh=mesh,
        scratch_types=[
            pltpu.VMEM((NLEV * NPAN * RB, 128), jnp.float32),
            pltpu.VMEM((128,), jnp.float32),
            pltpu.SemaphoreType.DMA,
            pltpu.SemaphoreType.DMA,
        ],
        compiler_params=pltpu.CompilerParams(use_tc_tiling_on_sc=True),
    )


def _tc_stencil_body(h_ref, out_ref):
    lvl = pl.program_id(0) + NLEV
    R = jnp.where(lvl == 1, jnp.float32(8.0), jnp.float32(16.0))
    h = h_ref[0, 0, 0]
    c = h[1:-1, 1:-1]
    m = ((c > h[:-2, :-2]) & (c > h[:-2, 1:-1]) & (c > h[:-2, 2:])
         & (c > h[1:-1, :-2]) & (c > h[1:-1, 2:])
         & (c > h[2:, :-2]) & (c > h[2:, 1:-1]) & (c > h[2:, 2:]))
    mf = m.astype(jnp.float32)
    ii = (lax.broadcasted_iota(jnp.int32, (H - 2, W - 2), 0)
          .astype(jnp.float32) + 1.0)
    jj = (lax.broadcasted_iota(jnp.int32, (H - 2, W - 2), 1)
          .astype(jnp.float32) + 1.0)
    si = jnp.sum(mf * ii) * R
    sj = jnp.sum(mf * jj) * R
    cnt = jnp.sum(mf)
    lane = lax.broadcasted_iota(jnp.int32, (1, 1, 1, 128), 3)
    out_ref[...] = jnp.where(
        lane == 0, si, jnp.where(lane == 1, sj,
                                 jnp.where(lane == 2, cnt, 0.0)))


def _tc_stencil(pre_offset):
    return pl.pallas_call(
        _tc_stencil_body,
        grid=(NTCLEV, NBATCH),
        in_specs=[pl.BlockSpec((1, 1, 1, H, W),
                               lambda l, i: (l + NLEV, i, 2, 0, 0))],
        out_specs=pl.BlockSpec((1, 1, 1, 128), lambda l, i: (l, i, 0, 0)),
        out_shape=jax.ShapeDtypeStruct((NTCLEV, NBATCH, 1, 128),
                                       jnp.float32),
    )(pre_offset)


def _tc_combine_body(part_ref, tcp_ref, t_ref, out_ref):
    p = part_ref[...]
    tcp = tcp_ref[...]
    t = t_ref[...]
    seg = lax.broadcasted_iota(jnp.int32, (NW, 128), 1) // 16
    grp = lax.broadcasted_iota(jnp.int32, (NW, 128), 0) // NSTRIP
    tlane = lax.broadcasted_iota(jnp.int32, (NTCLEV * NBATCH, 128), 1)
    trow = lax.broadcasted_iota(jnp.int32, (NTCLEV * NBATCH, 128), 0) % NBATCH
    nrow = lax.broadcasted_iota(jnp.int32, (NBATCH, 200), 0)
    cx = (t[:, :, 0] + t[:, :, 2]) * 0.5
    cy = (t[:, :, 1] + t[:, :, 3]) * 0.5

    zero = jnp.float32(0.0)
    off_x = zero
    off_y = zero
    cs_tx = zero
    cs_ty = zero
    ts_tx = zero
    ts_ty = zero
    point_sum = zero
    for nn in range(NBATCH):
        mrow = grp == nn
        mtrow = trow == nn
        si_n = (jnp.sum(jnp.where(mrow & (seg == 0), p, 0.0))
                + jnp.sum(jnp.where(mtrow & (tlane == 0), tcp, 0.0)))
        sj_n = (jnp.sum(jnp.where(mrow & (seg == 1), p, 0.0))
                + jnp.sum(jnp.where(mtrow & (tlane == 1), tcp, 0.0)))
        c_n = (jnp.sum(jnp.where(mrow & (seg == 2), p, 0.0))
               + jnp.sum(jnp.where(mtrow & (tlane == 2), tcp, 0.0)))
        tx_n = jnp.sum(jnp.where(nrow == nn, cx, 0.0))
        ty_n = jnp.sum(jnp.where(nrow == nn, cy, 0.0))
        dx = jnp.abs(si_n - tx_n)
        dy = jnp.abs(sj_n - ty_n)
        off_x = off_x + jnp.where(dx < 1.0, 0.5 * dx * dx, dx - 0.5)
        off_y = off_y + jnp.where(dy < 1.0, 0.5 * dy * dy, dy - 0.5)
        cs_tx = cs_tx + si_n
        cs_ty = cs_ty + sj_n
        ts_tx = ts_tx + tx_n
        ts_ty = ts_ty + ty_n
        point_sum = point_sum + c_n
    loss = (off_x / jnp.abs(off_x) * (cs_tx - ts_tx)
            + off_y / jnp.abs(off_y) * (cs_ty - ts_ty)) / point_sum
    out_ref[0, 0] = loss


def _tc_combine(part, tcp, target):
    return pl.pallas_call(
        _tc_combine_body,
        out_shape=jax.ShapeDtypeStruct((1, 1), jnp.float32),
        out_specs=pl.BlockSpec(memory_space=pltpu.SMEM),
    )(part, tcp, target)


def kernel(target, pre_offset):
    part = _make_sc_partials()(pre_offset)
    tcp = _tc_stencil(pre_offset)
    loss = _tc_combine(part, tcp.reshape(NTCLEV * NBATCH, 128), target)
    return loss[0, 0]
